# baseline (speedup 1.0000x reference)
import functools
import math

import jax
import jax.numpy as jnp
from jax.experimental import pallas as pl
from jax.experimental.pallas import tpu as pltpu

CUTOFF = 0.5
BLK_I = 256
BLK_J = 1024


def _prologue_kernel(scal_ref, z_ref, w_ref, b_ref, dd_ref, h_ref, init_ref):
    inv_mu = scal_ref[0, 0]
    neg_c2 = scal_ref[0, 1]
    h = (
        jnp.dot(z_ref[...], w_ref[...], preferred_element_type=jnp.float32)
        + b_ref[...]
    )
    h_ref[...] = h
    dd = dd_ref[...]
    delta = 1.0 / dd - inv_mu
    sens = jnp.exp2(delta * delta * neg_c2)
    coef = jnp.where(dd < CUTOFF, sens, 0.0)
    init_ref[...] = -coef * h


def _agg_kernel(scal_ref, dist_ref, h_ref, init_ref, out_ref):
    j = pl.program_id(1)
    inv_mu = scal_ref[0, 0]
    neg_c2 = scal_ref[0, 1]
    d = dist_ref[...]
    delta = 1.0 / d - inv_mu
    sens = jnp.exp2(delta * delta * neg_c2)
    w = jnp.where(d < CUTOFF, sens, 0.0)
    part = jnp.dot(w, h_ref[...], preferred_element_type=jnp.float32)

    @pl.when(j == 0)
    def _init():
        out_ref[...] = init_ref[...] + part

    @pl.when(j != 0)
    def _acc():
        out_ref[...] += part


@functools.partial(jax.jit, static_argnames=())
def kernel(z, dist_matrix, W, B, mu, sigma):
    n, d_in = z.shape
    d_out = W.shape[1]

    inv_mu = 1.0 / mu[0]
    neg_c2 = -math.log2(math.e) / (2.0 * sigma[0] * sigma[0])
    scal = jnp.stack([inv_mu, neg_c2]).reshape(1, 2)

    dd = jnp.diagonal(dist_matrix).reshape(n, 1)

    h, init = pl.pallas_call(
        _prologue_kernel,
        out_shape=(
            jax.ShapeDtypeStruct((n, d_out), jnp.float32),
            jax.ShapeDtypeStruct((n, d_out), jnp.float32),
        ),
    )(scal, z, W, B.reshape(1, d_out), dd)

    grid = (n // BLK_I, n // BLK_J)
    out = pl.pallas_call(
        _agg_kernel,
        grid=grid,
        in_specs=[
            pl.BlockSpec((1, 2), lambda i, j: (0, 0)),
            pl.BlockSpec((BLK_I, BLK_J), lambda i, j: (i, j)),
            pl.BlockSpec((BLK_J, d_out), lambda i, j: (j, 0)),
            pl.BlockSpec((BLK_I, d_out), lambda i, j: (i, 0)),
        ],
        out_specs=pl.BlockSpec((BLK_I, d_out), lambda i, j: (i, 0)),
        out_shape=jax.ShapeDtypeStruct((n, d_out), jnp.float32),
        compiler_params=pltpu.CompilerParams(
            dimension_semantics=("parallel", "arbitrary"),
        ),
    )(scal, dist_matrix, h, init)
    return out


# speedup vs baseline: 1.1697x; 1.1697x over previous
import functools
import math

import jax
import jax.numpy as jnp
from jax.experimental import pallas as pl
from jax.experimental.pallas import tpu as pltpu

CUTOFF = 0.5
BLK_I = 256
BLK_J = 1024


def _prologue_kernel(scal_ref, z_ref, w_ref, b_ref, dblk_ref, h_ref, init_ref):
    inv_mu = scal_ref[0, 0]
    neg_c2 = scal_ref[0, 1]
    h = (
        jnp.dot(z_ref[...], w_ref[...], preferred_element_type=jnp.float32)
        + b_ref[...]
    )
    h_ref[...] = h
    db = dblk_ref[...]
    eye = (
        jax.lax.broadcasted_iota(jnp.int32, db.shape, 0)
        == jax.lax.broadcasted_iota(jnp.int32, db.shape, 1)
    )
    dd = jnp.sum(jnp.where(eye, db, 0.0), axis=1, keepdims=True)
    delta = 1.0 / dd - inv_mu
    sens = jnp.exp2(delta * delta * neg_c2)
    coef = jnp.where(dd < CUTOFF, sens, 0.0)
    init_ref[...] = -coef * h


def _agg_kernel(scal_ref, dist_ref, h_ref, init_ref, out_ref):
    j = pl.program_id(1)
    inv_mu = scal_ref[0, 0]
    neg_c2 = scal_ref[0, 1]
    d = dist_ref[...]
    delta = 1.0 / d - inv_mu
    sens = jnp.exp2(delta * delta * neg_c2)
    w = jnp.where(d < CUTOFF, sens, 0.0)
    part = jnp.dot(w, h_ref[...], preferred_element_type=jnp.float32)

    @pl.when(j == 0)
    def _init():
        out_ref[...] = init_ref[...] + part

    @pl.when(j != 0)
    def _acc():
        out_ref[...] += part


@functools.partial(jax.jit, static_argnames=())
def kernel(z, dist_matrix, W, B, mu, sigma):
    n, d_in = z.shape
    d_out = W.shape[1]

    inv_mu = 1.0 / mu[0]
    neg_c2 = -math.log2(math.e) / (2.0 * sigma[0] * sigma[0])
    scal = jnp.stack([inv_mu, neg_c2]).reshape(1, 2)

    h, init = pl.pallas_call(
        _prologue_kernel,
        grid=(n // BLK_I,),
        in_specs=[
            pl.BlockSpec((1, 2), lambda i: (0, 0)),
            pl.BlockSpec((BLK_I, d_in), lambda i: (i, 0)),
            pl.BlockSpec((d_in, d_out), lambda i: (0, 0)),
            pl.BlockSpec((1, d_out), lambda i: (0, 0)),
            pl.BlockSpec((BLK_I, BLK_I), lambda i: (i, i)),
        ],
        out_specs=(
            pl.BlockSpec((BLK_I, d_out), lambda i: (i, 0)),
            pl.BlockSpec((BLK_I, d_out), lambda i: (i, 0)),
        ),
        out_shape=(
            jax.ShapeDtypeStruct((n, d_out), jnp.float32),
            jax.ShapeDtypeStruct((n, d_out), jnp.float32),
        ),
    )(scal, z, W, B.reshape(1, d_out), dist_matrix)

    grid = (n // BLK_I, n // BLK_J)
    out = pl.pallas_call(
        _agg_kernel,
        grid=grid,
        in_specs=[
            pl.BlockSpec((1, 2), lambda i, j: (0, 0)),
            pl.BlockSpec((BLK_I, BLK_J), lambda i, j: (i, j)),
            pl.BlockSpec((BLK_J, d_out), lambda i, j: (j, 0)),
            pl.BlockSpec((BLK_I, d_out), lambda i, j: (i, 0)),
        ],
        out_specs=pl.BlockSpec((BLK_I, d_out), lambda i, j: (i, 0)),
        out_shape=jax.ShapeDtypeStruct((n, d_out), jnp.float32),
        compiler_params=pltpu.CompilerParams(
            dimension_semantics=("parallel", "arbitrary"),
        ),
    )(scal, dist_matrix, h, init)
    return out


# speedup vs baseline: 2.6418x; 2.2586x over previous
import functools
import math

import jax
import jax.numpy as jnp
from jax.experimental import pallas as pl
from jax.experimental.pallas import tpu as pltpu

CUTOFF = 0.5
BLK_I = 256
N_FIXED = 4096


def _fused_kernel(scal_ref, z_ref, w_ref, b_ref, dblk_ref, dist_ref, out_ref,
                  h_scr):
    i = pl.program_id(0)
    inv_mu = scal_ref[0, 0]
    neg_c2 = scal_ref[0, 1]

    @pl.when(i == 0)
    def _compute_h():
        h_scr[...] = (
            jnp.dot(z_ref[...], w_ref[...], preferred_element_type=jnp.float32)
            + b_ref[...]
        )

    d = dist_ref[...]
    delta = 1.0 / d - inv_mu
    sens = jnp.exp2(delta * delta * neg_c2)
    w = jnp.where(d < CUTOFF, sens, 0.0)
    part = jnp.dot(w, h_scr[...], preferred_element_type=jnp.float32)

    db = dblk_ref[...]
    eye = (
        jax.lax.broadcasted_iota(jnp.int32, db.shape, 0)
        == jax.lax.broadcasted_iota(jnp.int32, db.shape, 1)
    )
    dd = jnp.sum(jnp.where(eye, db, 0.0), axis=1, keepdims=True)
    ddelta = 1.0 / dd - inv_mu
    dsens = jnp.exp2(ddelta * ddelta * neg_c2)
    coef = jnp.where(dd < CUTOFF, dsens, 0.0)
    h_rows = h_scr[pl.ds(i * BLK_I, BLK_I), :]
    out_ref[...] = part - coef * h_rows


@functools.partial(jax.jit, static_argnames=())
def kernel(z, dist_matrix, W, B, mu, sigma):
    n, d_in = z.shape
    d_out = W.shape[1]

    inv_mu = 1.0 / mu[0]
    neg_c2 = -math.log2(math.e) / (2.0 * sigma[0] * sigma[0])
    scal = jnp.stack([inv_mu, neg_c2]).reshape(1, 2)

    out = pl.pallas_call(
        _fused_kernel,
        grid=(n // BLK_I,),
        in_specs=[
            pl.BlockSpec((1, 2), lambda i: (0, 0)),
            pl.BlockSpec((n, d_in), lambda i: (0, 0)),
            pl.BlockSpec((d_in, d_out), lambda i: (0, 0)),
            pl.BlockSpec((1, d_out), lambda i: (0, 0)),
            pl.BlockSpec((BLK_I, BLK_I), lambda i: (i, i)),
            pl.BlockSpec((BLK_I, n), lambda i: (i, 0)),
        ],
        out_specs=pl.BlockSpec((BLK_I, d_out), lambda i: (i, 0)),
        out_shape=jax.ShapeDtypeStruct((n, d_out), jnp.float32),
        scratch_shapes=[pltpu.VMEM((n, d_out), jnp.float32)],
        compiler_params=pltpu.CompilerParams(
            dimension_semantics=("arbitrary",),
        ),
    )(scal, z, W, B.reshape(1, d_out), dist_matrix, dist_matrix)
    return out


# speedup vs baseline: 2.6852x; 1.0164x over previous
import functools
import math

import jax
import jax.numpy as jnp
from jax.experimental import pallas as pl
from jax.experimental.pallas import tpu as pltpu

CUTOFF = 0.5
BLK_I = 256
N_FIXED = 4096


def _fused_kernel(scal_ref, z_ref, w_ref, b_ref, dist_ref, out_ref, h_scr):
    i = pl.program_id(0)
    inv_mu = scal_ref[0, 0]
    neg_c2 = scal_ref[0, 1]

    @pl.when(i == 0)
    def _compute_h():
        h_scr[...] = (
            jnp.dot(z_ref[...], w_ref[...], preferred_element_type=jnp.float32)
            + b_ref[...]
        )

    d = dist_ref[...]
    delta = 1.0 / d - inv_mu
    sens = jnp.exp2(delta * delta * neg_c2)
    w = jnp.where(d < CUTOFF, sens, 0.0)
    part = jnp.dot(w, h_scr[...], preferred_element_type=jnp.float32)

    db = dist_ref[:, pl.ds(i * BLK_I, BLK_I)]
    eye = (
        jax.lax.broadcasted_iota(jnp.int32, db.shape, 0)
        == jax.lax.broadcasted_iota(jnp.int32, db.shape, 1)
    )
    dd = jnp.sum(jnp.where(eye, db, 0.0), axis=1, keepdims=True)
    ddelta = 1.0 / dd - inv_mu
    dsens = jnp.exp2(ddelta * ddelta * neg_c2)
    coef = jnp.where(dd < CUTOFF, dsens, 0.0)
    h_rows = h_scr[pl.ds(i * BLK_I, BLK_I), :]
    out_ref[...] = part - coef * h_rows


@functools.partial(jax.jit, static_argnames=())
def kernel(z, dist_matrix, W, B, mu, sigma):
    n, d_in = z.shape
    d_out = W.shape[1]

    inv_mu = 1.0 / mu[0]
    neg_c2 = -math.log2(math.e) / (2.0 * sigma[0] * sigma[0])
    scal = jnp.stack([inv_mu, neg_c2]).reshape(1, 2)

    out = pl.pallas_call(
        _fused_kernel,
        grid=(n // BLK_I,),
        in_specs=[
            pl.BlockSpec((1, 2), lambda i: (0, 0)),
            pl.BlockSpec((n, d_in), lambda i: (0, 0)),
            pl.BlockSpec((d_in, d_out), lambda i: (0, 0)),
            pl.BlockSpec((1, d_out), lambda i: (0, 0)),
            pl.BlockSpec((BLK_I, n), lambda i: (i, 0)),
        ],
        out_specs=pl.BlockSpec((BLK_I, d_out), lambda i: (i, 0)),
        out_shape=jax.ShapeDtypeStruct((n, d_out), jnp.float32),
        scratch_shapes=[pltpu.VMEM((n, d_out), jnp.float32)],
        compiler_params=pltpu.CompilerParams(
            dimension_semantics=("arbitrary",),
        ),
    )(scal, z, W, B.reshape(1, d_out), dist_matrix)
    return out
